# baseline (speedup 1.0000x reference)
import functools

import jax
import jax.numpy as jnp
from jax import lax
from jax.experimental import pallas as pl
from jax.experimental.pallas import tpu as pltpu
from jax.experimental.pallas import tpu_sc as plsc

N_USER = 27094
M_SPOT = 42852
E = 1000000
D = 64
N_LAYERS = 3

NC = 2
NS = 16
BLK = 128
SLAB = 16
NSLAB = 31
Q = SLAB * NSLAB
NBUF = 6
FIRE = 3
E_PAD = NS * BLK * Q
assert E_PAD >= E
NROW = E_PAD // BLK

NP_U = 27136
NP_S = 42880
ZR_U = NP_U // NS
ZR_S = NP_S // NS

_mesh = lambda: plsc.VectorSubcoreMesh(
    core_axis_name="c", subcore_axis_name="s", num_cores=NC, num_subcores=NS)


@functools.partial(
    pl.kernel,
    out_type=jax.ShapeDtypeStruct((2 * NP_S,), jnp.float32),
    mesh=_mesh(),
    scratch_types=[
        pltpu.VMEM((SLAB, BLK), jnp.int32),
        pltpu.VMEM((BLK,), jnp.float32),
        pltpu.VMEM((ZR_S,), jnp.float32),
        pltpu.VMEM_SHARED((NP_S,), jnp.float32),
        pltpu.SemaphoreType.DMA((SLAB,)),
    ],
    compiler_params=pltpu.CompilerParams(use_tc_tiling_on_sc=False),
)
def _degrees(uidx, sidx, zeros1, out, idxv, onesv, bounce, acc, sems):
  c = lax.axis_index("c")
  s = lax.axis_index("s")
  for j in range(BLK // 16):
    onesv[pl.ds(j * 16, 16)] = jnp.full((16,), 1.0, jnp.float32)
  pltpu.sync_copy(zeros1.at[pl.ds(s * ZR_S, ZR_S)], bounce)
  pltpu.sync_copy(bounce, acc.at[pl.ds(s * ZR_S, ZR_S)])
  plsc.subcore_barrier()

  def run(ix2d):
    def slab(t, carry):
      row0 = s * Q + t * SLAB
      pltpu.sync_copy(ix2d.at[pl.ds(row0, SLAB)], idxv)
      descs = []
      for j in range(SLAB):
        descs.append(
            pltpu.async_copy(onesv, acc.at[idxv.at[j]], sems.at[j], add=True))
      for d in descs:
        d.wait()
      return carry

    lax.fori_loop(0, NSLAB, slab, 0)

  @pl.when(c == 0)
  def _():
    run(uidx)

  @pl.when(c == 1)
  def _():
    run(sidx)

  plsc.subcore_barrier()
  pltpu.sync_copy(acc.at[pl.ds(s * ZR_S, ZR_S)], bounce)
  pltpu.sync_copy(bounce, out.at[pl.ds(c * NP_S + s * ZR_S, ZR_S)])


def _make_pass(nsrc_pad, ndst_pad):
  zr = ndst_pad // NS
  nch = 16 if zr % 16 == 0 else 20
  assert zr % nch == 0
  ch = zr // nch

  @functools.partial(
      pl.kernel,
      out_type=jax.ShapeDtypeStruct((2 * ndst_pad, 32), jnp.float32),
      mesh=_mesh(),
      scratch_types=[
          pltpu.VMEM((SLAB, BLK), jnp.int32),
          pltpu.VMEM((SLAB, BLK), jnp.int32),
          pltpu.VMEM((NBUF, BLK, 32), jnp.float32),
          pltpu.VMEM((ch, 32), jnp.float32),
          pltpu.VMEM_SHARED((ndst_pad, 32), jnp.float32),
          pltpu.SemaphoreType.DMA((NBUF,)),
          pltpu.SemaphoreType.DMA((NBUF,)),
      ],
      compiler_params=pltpu.CompilerParams(use_tc_tiling_on_sc=False),
  )
  def _pass(tlo, thi, src_ix, dst_ix, zeros2, out, sv, dv, rows, bounce, acc,
            gsem, ssem):
    c = lax.axis_index("c")
    s = lax.axis_index("s")
    for t in range(nch):
      off = s * zr + t * ch
      pltpu.sync_copy(zeros2.at[pl.ds(off, ch)], bounce)
      pltpu.sync_copy(bounce, acc.at[pl.ds(off, ch)])
    plsc.subcore_barrier()

    def run(tbl):
      def slab(t, carry):
        row0 = s * Q + t * SLAB
        pltpu.sync_copy(src_ix.at[pl.ds(row0, SLAB)], sv)
        pltpu.sync_copy(dst_ix.at[pl.ds(row0, SLAB)], dv)
        gd = {}
        sd = {}
        for j in range(FIRE):
          gd[j] = pltpu.async_copy(tbl.at[sv.at[j]], rows.at[j], gsem.at[j])
        for j in range(SLAB):
          b = j % NBUF
          gd[j].wait()
          sd[j] = pltpu.async_copy(rows.at[b], acc.at[dv.at[j]], ssem.at[b],
                                   add=True)
          jn = j + FIRE
          if jn < SLAB:
            bn = jn % NBUF
            if jn >= NBUF:
              sd[jn - NBUF].wait()
            gd[jn] = pltpu.async_copy(tbl.at[sv.at[jn]], rows.at[bn],
                                      gsem.at[bn])
        for j in range(SLAB - NBUF, SLAB):
          sd[j].wait()
        return carry

      lax.fori_loop(0, NSLAB, slab, 0)

    @pl.when(c == 0)
    def _():
      run(tlo)

    @pl.when(c == 1)
    def _():
      run(thi)

    plsc.subcore_barrier()
    for t in range(nch):
      off = s * zr + t * ch
      pltpu.sync_copy(acc.at[pl.ds(off, ch)], bounce)
      pltpu.sync_copy(bounce, out.at[pl.ds(c * ndst_pad + off, ch)])

  return _pass


_pass_to_user = _make_pass(NP_S, NP_U)
_pass_to_spot = _make_pass(NP_U, NP_S)


def _tail_body(do_cross, x0_ref, rlo_ref, rhi_ref, c_ref, w_ref, b_ref, o_ref):
  r = jnp.concatenate([rlo_ref[...], rhi_ref[...]], axis=1)
  x0 = 0.25 * (x0_ref[...] + c_ref[...] * r)
  x = x0
  if do_cross:
    w = w_ref[...]
    bb = b_ref[...]
    for i in range(N_LAYERS):
      t = jnp.sum(x * x0, axis=1, keepdims=True)
      x = t * w[i][None, :] + bb[i][None, :] + x
      x = x * lax.rsqrt(jnp.sum(x * x, axis=1, keepdims=True))
  o_ref[...] = x


def _tail(x0p, rlo, rhi, cp, Ws, bs, do_cross):
  npad = x0p.shape[0]
  bm = 128
  grid = (npad // bm,)
  return pl.pallas_call(
      functools.partial(_tail_body, do_cross),
      grid=grid,
      in_specs=[
          pl.BlockSpec((bm, D), lambda i: (i, 0)),
          pl.BlockSpec((bm, 32), lambda i: (i, 0)),
          pl.BlockSpec((bm, 32), lambda i: (i, 0)),
          pl.BlockSpec((bm, 1), lambda i: (i, 0)),
          pl.BlockSpec((N_LAYERS, D), lambda i: (0, 0)),
          pl.BlockSpec((N_LAYERS, D), lambda i: (0, 0)),
      ],
      out_specs=pl.BlockSpec((bm, D), lambda i: (i, 0)),
      out_shape=jax.ShapeDtypeStruct((npad, D), jnp.float32),
  )(x0p, rlo, rhi, cp, Ws, bs)


def kernel(user_emb, spot_emb, user_idx, spot_idx, Ws, bs):
  f32 = jnp.float32
  pad_r = jnp.arange(E_PAD - E, dtype=jnp.int32) % 8
  uix = jnp.concatenate([user_idx, N_USER + pad_r]).reshape(NROW, BLK)
  six = jnp.concatenate([spot_idx, M_SPOT + pad_r]).reshape(NROW, BLK)

  zeros1 = jnp.zeros((NP_S,), f32)
  zeros2u = jnp.zeros((NP_U, 32), f32)
  zeros2s = jnp.zeros((NP_S, 32), f32)

  hist = _degrees(uix, six, zeros1)
  du = jnp.maximum(hist[:N_USER], 1.0)
  ds = jnp.maximum(hist[NP_S:NP_S + M_SPOT], 1.0)
  cu = lax.rsqrt(du)
  cs = lax.rsqrt(ds)
  cu2p = jnp.pad(1.0 / du, (0, NP_U - N_USER))[:, None]
  cs2p = jnp.pad(1.0 / ds, (0, NP_S - M_SPOT))[:, None]

  su = user_emb * cu[:, None]
  ss = spot_emb * cs[:, None]
  ulo = jnp.pad(su[:, :32], ((0, NP_U - N_USER), (0, 0)))
  uhi = jnp.pad(su[:, 32:], ((0, NP_U - N_USER), (0, 0)))
  slo = jnp.pad(ss[:, :32], ((0, NP_S - M_SPOT), (0, 0)))
  shi = jnp.pad(ss[:, 32:], ((0, NP_S - M_SPOT), (0, 0)))

  ru_sum = jnp.zeros((2, NP_U, 32), f32)
  rs_sum = jnp.zeros((2, NP_S, 32), f32)
  for _ in range(N_LAYERS):
    ru = _pass_to_user(slo, shi, six, uix, zeros2u).reshape(2, NP_U, 32)
    rs = _pass_to_spot(ulo, uhi, uix, six, zeros2s).reshape(2, NP_S, 32)
    ru_sum = ru_sum + ru
    rs_sum = rs_sum + rs
    ulo = ru[0] * cu2p
    uhi = ru[1] * cu2p
    slo = rs[0] * cs2p
    shi = rs[1] * cs2p

  cup = jnp.pad(cu, (0, NP_U - N_USER))[:, None]
  csp = jnp.pad(cs, (0, NP_S - M_SPOT))[:, None]
  u0p = jnp.pad(user_emb, ((0, NP_U - N_USER), (0, 0)))
  s0p = jnp.pad(spot_emb, ((0, NP_S - M_SPOT), (0, 0)))

  user_final = _tail(u0p, ru_sum[0], ru_sum[1], cup, Ws, bs, False)[:N_USER]
  x = _tail(s0p, rs_sum[0], rs_sum[1], csp, Ws, bs, True)[:M_SPOT]
  return jnp.concatenate([x, user_final], axis=0)


# speedup vs baseline: 1.0547x; 1.0547x over previous
import functools

import jax
import jax.numpy as jnp
from jax import lax
from jax.experimental import pallas as pl
from jax.experimental.pallas import tpu as pltpu
from jax.experimental.pallas import tpu_sc as plsc

N_USER = 27094
M_SPOT = 42852
E = 1000000
D = 64
N_LAYERS = 3

NC = 2
NS = 16
BLK = 128
SLAB = 16
NSLAB = 31
Q = SLAB * NSLAB
NBUF = 6
FIRE = 4
E_PAD = NS * BLK * Q
assert E_PAD >= E
NROW = E_PAD // BLK

NP_U = 27136
NP_S = 42880
ZR_U = NP_U // NS
ZR_S = NP_S // NS

_mesh = lambda: plsc.VectorSubcoreMesh(
    core_axis_name="c", subcore_axis_name="s", num_cores=NC, num_subcores=NS)


@functools.partial(
    pl.kernel,
    out_type=jax.ShapeDtypeStruct((2 * NP_S,), jnp.float32),
    mesh=_mesh(),
    scratch_types=[
        pltpu.VMEM((SLAB, BLK), jnp.int32),
        pltpu.VMEM((BLK,), jnp.float32),
        pltpu.VMEM((ZR_S,), jnp.float32),
        pltpu.VMEM_SHARED((NP_S,), jnp.float32),
        pltpu.SemaphoreType.DMA((SLAB,)),
    ],
    compiler_params=pltpu.CompilerParams(use_tc_tiling_on_sc=False),
)
def _degrees(uidx, sidx, zeros1, out, idxv, onesv, bounce, acc, sems):
  c = lax.axis_index("c")
  s = lax.axis_index("s")
  for j in range(BLK // 16):
    onesv[pl.ds(j * 16, 16)] = jnp.full((16,), 1.0, jnp.float32)
  pltpu.sync_copy(zeros1.at[pl.ds(s * ZR_S, ZR_S)], bounce)
  pltpu.sync_copy(bounce, acc.at[pl.ds(s * ZR_S, ZR_S)])
  plsc.subcore_barrier()

  def run(ix2d):
    def slab(t, carry):
      row0 = s * Q + t * SLAB
      pltpu.sync_copy(ix2d.at[pl.ds(row0, SLAB)], idxv)
      descs = []
      for j in range(SLAB):
        descs.append(
            pltpu.async_copy(onesv, acc.at[idxv.at[j]], sems.at[j], add=True))
      for d in descs:
        d.wait()
      return carry

    lax.fori_loop(0, NSLAB, slab, 0)

  @pl.when(c == 0)
  def _():
    run(uidx)

  @pl.when(c == 1)
  def _():
    run(sidx)

  plsc.subcore_barrier()
  pltpu.sync_copy(acc.at[pl.ds(s * ZR_S, ZR_S)], bounce)
  pltpu.sync_copy(bounce, out.at[pl.ds(c * NP_S + s * ZR_S, ZR_S)])


def _make_pass(nsrc_pad, ndst_pad):
  zr = ndst_pad // NS
  nch = 16 if zr % 16 == 0 else 20
  assert zr % nch == 0
  ch = zr // nch

  @functools.partial(
      pl.kernel,
      out_type=jax.ShapeDtypeStruct((2 * ndst_pad, 32), jnp.float32),
      mesh=_mesh(),
      scratch_types=[
          pltpu.VMEM((SLAB, BLK), jnp.int32),
          pltpu.VMEM((SLAB, BLK), jnp.int32),
          pltpu.VMEM((NBUF, BLK, 32), jnp.float32),
          pltpu.VMEM((ch, 32), jnp.float32),
          pltpu.VMEM_SHARED((ndst_pad, 32), jnp.float32),
          pltpu.SemaphoreType.DMA((NBUF,)),
          pltpu.SemaphoreType.DMA((NBUF,)),
      ],
      compiler_params=pltpu.CompilerParams(use_tc_tiling_on_sc=False),
  )
  def _pass(tlo, thi, src_ix, dst_ix, zeros2, out, sv, dv, rows, bounce, acc,
            gsem, ssem):
    c = lax.axis_index("c")
    s = lax.axis_index("s")
    for t in range(nch):
      off = s * zr + t * ch
      pltpu.sync_copy(zeros2.at[pl.ds(off, ch)], bounce)
      pltpu.sync_copy(bounce, acc.at[pl.ds(off, ch)])
    plsc.subcore_barrier()

    def run(tbl):
      def slab(t, carry):
        row0 = s * Q + t * SLAB
        pltpu.sync_copy(src_ix.at[pl.ds(row0, SLAB)], sv)
        pltpu.sync_copy(dst_ix.at[pl.ds(row0, SLAB)], dv)
        gd = {}
        sd = {}
        for j in range(FIRE):
          gd[j] = pltpu.async_copy(tbl.at[sv.at[j]], rows.at[j], gsem.at[j])
        for j in range(SLAB):
          b = j % NBUF
          gd[j].wait()
          sd[j] = pltpu.async_copy(rows.at[b], acc.at[dv.at[j]], ssem.at[b],
                                   add=True)
          jn = j + FIRE
          if jn < SLAB:
            bn = jn % NBUF
            if jn >= NBUF:
              sd[jn - NBUF].wait()
            gd[jn] = pltpu.async_copy(tbl.at[sv.at[jn]], rows.at[bn],
                                      gsem.at[bn])
        for j in range(SLAB - NBUF, SLAB):
          sd[j].wait()
        return carry

      lax.fori_loop(0, NSLAB, slab, 0)

    @pl.when(c == 0)
    def _():
      run(tlo)

    @pl.when(c == 1)
    def _():
      run(thi)

    plsc.subcore_barrier()
    for t in range(nch):
      off = s * zr + t * ch
      pltpu.sync_copy(acc.at[pl.ds(off, ch)], bounce)
      pltpu.sync_copy(bounce, out.at[pl.ds(c * ndst_pad + off, ch)])

  return _pass


_pass_to_user = _make_pass(NP_S, NP_U)
_pass_to_spot = _make_pass(NP_U, NP_S)


def _tail_body(do_cross, x0_ref, rlo_ref, rhi_ref, c_ref, w_ref, b_ref, o_ref):
  r = jnp.concatenate([rlo_ref[...], rhi_ref[...]], axis=1)
  x0 = 0.25 * (x0_ref[...] + c_ref[...] * r)
  x = x0
  if do_cross:
    w = w_ref[...]
    bb = b_ref[...]
    for i in range(N_LAYERS):
      t = jnp.sum(x * x0, axis=1, keepdims=True)
      x = t * w[i][None, :] + bb[i][None, :] + x
      x = x * lax.rsqrt(jnp.sum(x * x, axis=1, keepdims=True))
  o_ref[...] = x


def _tail(x0p, rlo, rhi, cp, Ws, bs, do_cross):
  npad = x0p.shape[0]
  bm = 128
  grid = (npad // bm,)
  return pl.pallas_call(
      functools.partial(_tail_body, do_cross),
      grid=grid,
      in_specs=[
          pl.BlockSpec((bm, D), lambda i: (i, 0)),
          pl.BlockSpec((bm, 32), lambda i: (i, 0)),
          pl.BlockSpec((bm, 32), lambda i: (i, 0)),
          pl.BlockSpec((bm, 1), lambda i: (i, 0)),
          pl.BlockSpec((N_LAYERS, D), lambda i: (0, 0)),
          pl.BlockSpec((N_LAYERS, D), lambda i: (0, 0)),
      ],
      out_specs=pl.BlockSpec((bm, D), lambda i: (i, 0)),
      out_shape=jax.ShapeDtypeStruct((npad, D), jnp.float32),
  )(x0p, rlo, rhi, cp, Ws, bs)


def kernel(user_emb, spot_emb, user_idx, spot_idx, Ws, bs):
  f32 = jnp.float32
  pad_r = jnp.arange(E_PAD - E, dtype=jnp.int32) % 8
  uix = jnp.concatenate([user_idx, N_USER + pad_r]).reshape(NROW, BLK)
  six = jnp.concatenate([spot_idx, M_SPOT + pad_r]).reshape(NROW, BLK)

  zeros1 = jnp.zeros((NP_S,), f32)
  zeros2u = jnp.zeros((NP_U, 32), f32)
  zeros2s = jnp.zeros((NP_S, 32), f32)

  hist = _degrees(uix, six, zeros1)
  du = jnp.maximum(hist[:N_USER], 1.0)
  ds = jnp.maximum(hist[NP_S:NP_S + M_SPOT], 1.0)
  cu = lax.rsqrt(du)
  cs = lax.rsqrt(ds)
  cu2p = jnp.pad(1.0 / du, (0, NP_U - N_USER))[:, None]
  cs2p = jnp.pad(1.0 / ds, (0, NP_S - M_SPOT))[:, None]

  su = user_emb * cu[:, None]
  ss = spot_emb * cs[:, None]
  ulo = jnp.pad(su[:, :32], ((0, NP_U - N_USER), (0, 0)))
  uhi = jnp.pad(su[:, 32:], ((0, NP_U - N_USER), (0, 0)))
  slo = jnp.pad(ss[:, :32], ((0, NP_S - M_SPOT), (0, 0)))
  shi = jnp.pad(ss[:, 32:], ((0, NP_S - M_SPOT), (0, 0)))

  ru_sum = jnp.zeros((2, NP_U, 32), f32)
  rs_sum = jnp.zeros((2, NP_S, 32), f32)
  for _ in range(N_LAYERS):
    ru = _pass_to_user(slo, shi, six, uix, zeros2u).reshape(2, NP_U, 32)
    rs = _pass_to_spot(ulo, uhi, uix, six, zeros2s).reshape(2, NP_S, 32)
    ru_sum = ru_sum + ru
    rs_sum = rs_sum + rs
    ulo = ru[0] * cu2p
    uhi = ru[1] * cu2p
    slo = rs[0] * cs2p
    shi = rs[1] * cs2p

  cup = jnp.pad(cu, (0, NP_U - N_USER))[:, None]
  csp = jnp.pad(cs, (0, NP_S - M_SPOT))[:, None]
  u0p = jnp.pad(user_emb, ((0, NP_U - N_USER), (0, 0)))
  s0p = jnp.pad(spot_emb, ((0, NP_S - M_SPOT), (0, 0)))

  user_final = _tail(u0p, ru_sum[0], ru_sum[1], cup, Ws, bs, False)[:N_USER]
  x = _tail(s0p, rs_sum[0], rs_sum[1], csp, Ws, bs, True)[:M_SPOT]
  return jnp.concatenate([x, user_final], axis=0)


# speedup vs baseline: 1.0608x; 1.0058x over previous
import functools

import jax
import jax.numpy as jnp
from jax import lax
from jax.experimental import pallas as pl
from jax.experimental.pallas import tpu as pltpu
from jax.experimental.pallas import tpu_sc as plsc

N_USER = 27094
M_SPOT = 42852
E = 1000000
D = 64
N_LAYERS = 3

NC = 2
NS = 16
BLK = 128
SLAB = 16
NSLAB = 31
Q = SLAB * NSLAB
NBUF = 6
FIRE = 5
E_PAD = NS * BLK * Q
assert E_PAD >= E
NROW = E_PAD // BLK

NP_U = 27136
NP_S = 42880
ZR_U = NP_U // NS
ZR_S = NP_S // NS

_mesh = lambda: plsc.VectorSubcoreMesh(
    core_axis_name="c", subcore_axis_name="s", num_cores=NC, num_subcores=NS)


@functools.partial(
    pl.kernel,
    out_type=jax.ShapeDtypeStruct((2 * NP_S,), jnp.float32),
    mesh=_mesh(),
    scratch_types=[
        pltpu.VMEM((SLAB, BLK), jnp.int32),
        pltpu.VMEM((BLK,), jnp.float32),
        pltpu.VMEM((ZR_S,), jnp.float32),
        pltpu.VMEM_SHARED((NP_S,), jnp.float32),
        pltpu.SemaphoreType.DMA((SLAB,)),
    ],
    compiler_params=pltpu.CompilerParams(use_tc_tiling_on_sc=False),
)
def _degrees(uidx, sidx, zeros1, out, idxv, onesv, bounce, acc, sems):
  c = lax.axis_index("c")
  s = lax.axis_index("s")
  for j in range(BLK // 16):
    onesv[pl.ds(j * 16, 16)] = jnp.full((16,), 1.0, jnp.float32)
  pltpu.sync_copy(zeros1.at[pl.ds(s * ZR_S, ZR_S)], bounce)
  pltpu.sync_copy(bounce, acc.at[pl.ds(s * ZR_S, ZR_S)])
  plsc.subcore_barrier()

  def run(ix2d):
    def slab(t, carry):
      row0 = s * Q + t * SLAB
      pltpu.sync_copy(ix2d.at[pl.ds(row0, SLAB)], idxv)
      descs = []
      for j in range(SLAB):
        descs.append(
            pltpu.async_copy(onesv, acc.at[idxv.at[j]], sems.at[j], add=True))
      for d in descs:
        d.wait()
      return carry

    lax.fori_loop(0, NSLAB, slab, 0)

  @pl.when(c == 0)
  def _():
    run(uidx)

  @pl.when(c == 1)
  def _():
    run(sidx)

  plsc.subcore_barrier()
  pltpu.sync_copy(acc.at[pl.ds(s * ZR_S, ZR_S)], bounce)
  pltpu.sync_copy(bounce, out.at[pl.ds(c * NP_S + s * ZR_S, ZR_S)])


def _make_pass(nsrc_pad, ndst_pad):
  zr = ndst_pad // NS
  nch = 16 if zr % 16 == 0 else 20
  assert zr % nch == 0
  ch = zr // nch

  @functools.partial(
      pl.kernel,
      out_type=jax.ShapeDtypeStruct((2 * ndst_pad, 32), jnp.float32),
      mesh=_mesh(),
      scratch_types=[
          pltpu.VMEM((SLAB, BLK), jnp.int32),
          pltpu.VMEM((SLAB, BLK), jnp.int32),
          pltpu.VMEM((NBUF, BLK, 32), jnp.float32),
          pltpu.VMEM((ch, 32), jnp.float32),
          pltpu.VMEM_SHARED((ndst_pad, 32), jnp.float32),
          pltpu.SemaphoreType.DMA((NBUF,)),
          pltpu.SemaphoreType.DMA((NBUF,)),
      ],
      compiler_params=pltpu.CompilerParams(use_tc_tiling_on_sc=False),
  )
  def _pass(tlo, thi, src_ix, dst_ix, zeros2, out, sv, dv, rows, bounce, acc,
            gsem, ssem):
    c = lax.axis_index("c")
    s = lax.axis_index("s")
    for t in range(nch):
      off = s * zr + t * ch
      pltpu.sync_copy(zeros2.at[pl.ds(off, ch)], bounce)
      pltpu.sync_copy(bounce, acc.at[pl.ds(off, ch)])
    plsc.subcore_barrier()

    def run(tbl):
      def slab(t, carry):
        row0 = s * Q + t * SLAB
        pltpu.sync_copy(src_ix.at[pl.ds(row0, SLAB)], sv)
        pltpu.sync_copy(dst_ix.at[pl.ds(row0, SLAB)], dv)
        gd = {}
        sd = {}
        for j in range(FIRE):
          gd[j] = pltpu.async_copy(tbl.at[sv.at[j]], rows.at[j], gsem.at[j])
        for j in range(SLAB):
          b = j % NBUF
          gd[j].wait()
          sd[j] = pltpu.async_copy(rows.at[b], acc.at[dv.at[j]], ssem.at[b],
                                   add=True)
          jn = j + FIRE
          if jn < SLAB:
            bn = jn % NBUF
            if jn >= NBUF:
              sd[jn - NBUF].wait()
            gd[jn] = pltpu.async_copy(tbl.at[sv.at[jn]], rows.at[bn],
                                      gsem.at[bn])
        for j in range(SLAB - NBUF, SLAB):
          sd[j].wait()
        return carry

      lax.fori_loop(0, NSLAB, slab, 0)

    @pl.when(c == 0)
    def _():
      run(tlo)

    @pl.when(c == 1)
    def _():
      run(thi)

    plsc.subcore_barrier()
    for t in range(nch):
      off = s * zr + t * ch
      pltpu.sync_copy(acc.at[pl.ds(off, ch)], bounce)
      pltpu.sync_copy(bounce, out.at[pl.ds(c * ndst_pad + off, ch)])

  return _pass


_pass_to_user = _make_pass(NP_S, NP_U)
_pass_to_spot = _make_pass(NP_U, NP_S)


def _tail_body(do_cross, x0_ref, rlo_ref, rhi_ref, c_ref, w_ref, b_ref, o_ref):
  r = jnp.concatenate([rlo_ref[...], rhi_ref[...]], axis=1)
  x0 = 0.25 * (x0_ref[...] + c_ref[...] * r)
  x = x0
  if do_cross:
    w = w_ref[...]
    bb = b_ref[...]
    for i in range(N_LAYERS):
      t = jnp.sum(x * x0, axis=1, keepdims=True)
      x = t * w[i][None, :] + bb[i][None, :] + x
      x = x * lax.rsqrt(jnp.sum(x * x, axis=1, keepdims=True))
  o_ref[...] = x


def _tail(x0p, rlo, rhi, cp, Ws, bs, do_cross):
  npad = x0p.shape[0]
  bm = 128
  grid = (npad // bm,)
  return pl.pallas_call(
      functools.partial(_tail_body, do_cross),
      grid=grid,
      in_specs=[
          pl.BlockSpec((bm, D), lambda i: (i, 0)),
          pl.BlockSpec((bm, 32), lambda i: (i, 0)),
          pl.BlockSpec((bm, 32), lambda i: (i, 0)),
          pl.BlockSpec((bm, 1), lambda i: (i, 0)),
          pl.BlockSpec((N_LAYERS, D), lambda i: (0, 0)),
          pl.BlockSpec((N_LAYERS, D), lambda i: (0, 0)),
      ],
      out_specs=pl.BlockSpec((bm, D), lambda i: (i, 0)),
      out_shape=jax.ShapeDtypeStruct((npad, D), jnp.float32),
  )(x0p, rlo, rhi, cp, Ws, bs)


def kernel(user_emb, spot_emb, user_idx, spot_idx, Ws, bs):
  f32 = jnp.float32
  pad_r = jnp.arange(E_PAD - E, dtype=jnp.int32) % 8
  uix = jnp.concatenate([user_idx, N_USER + pad_r]).reshape(NROW, BLK)
  six = jnp.concatenate([spot_idx, M_SPOT + pad_r]).reshape(NROW, BLK)

  zeros1 = jnp.zeros((NP_S,), f32)
  zeros2u = jnp.zeros((NP_U, 32), f32)
  zeros2s = jnp.zeros((NP_S, 32), f32)

  hist = _degrees(uix, six, zeros1)
  du = jnp.maximum(hist[:N_USER], 1.0)
  ds = jnp.maximum(hist[NP_S:NP_S + M_SPOT], 1.0)
  cu = lax.rsqrt(du)
  cs = lax.rsqrt(ds)
  cu2p = jnp.pad(1.0 / du, (0, NP_U - N_USER))[:, None]
  cs2p = jnp.pad(1.0 / ds, (0, NP_S - M_SPOT))[:, None]

  su = user_emb * cu[:, None]
  ss = spot_emb * cs[:, None]
  ulo = jnp.pad(su[:, :32], ((0, NP_U - N_USER), (0, 0)))
  uhi = jnp.pad(su[:, 32:], ((0, NP_U - N_USER), (0, 0)))
  slo = jnp.pad(ss[:, :32], ((0, NP_S - M_SPOT), (0, 0)))
  shi = jnp.pad(ss[:, 32:], ((0, NP_S - M_SPOT), (0, 0)))

  ru_sum = jnp.zeros((2, NP_U, 32), f32)
  rs_sum = jnp.zeros((2, NP_S, 32), f32)
  for _ in range(N_LAYERS):
    ru = _pass_to_user(slo, shi, six, uix, zeros2u).reshape(2, NP_U, 32)
    rs = _pass_to_spot(ulo, uhi, uix, six, zeros2s).reshape(2, NP_S, 32)
    ru_sum = ru_sum + ru
    rs_sum = rs_sum + rs
    ulo = ru[0] * cu2p
    uhi = ru[1] * cu2p
    slo = rs[0] * cs2p
    shi = rs[1] * cs2p

  cup = jnp.pad(cu, (0, NP_U - N_USER))[:, None]
  csp = jnp.pad(cs, (0, NP_S - M_SPOT))[:, None]
  u0p = jnp.pad(user_emb, ((0, NP_U - N_USER), (0, 0)))
  s0p = jnp.pad(spot_emb, ((0, NP_S - M_SPOT), (0, 0)))

  user_final = _tail(u0p, ru_sum[0], ru_sum[1], cup, Ws, bs, False)[:N_USER]
  x = _tail(s0p, rs_sum[0], rs_sum[1], csp, Ws, bs, True)[:M_SPOT]
  return jnp.concatenate([x, user_final], axis=0)
